# baseline (speedup 1.0000x reference)
import jax
import jax.numpy as jnp
from jax import lax
from jax.experimental import pallas as pl
from jax.experimental.pallas import tpu as pltpu
from jax.experimental.pallas import tpu_sc as plsc

N = 10000
E = 320000
NC = 2
NS = 16
NW = NC * NS
K = 80
NPAD = 10112
RPS = NPAD // NS
EPW = 10240
EPAD = NW * EPW


_SC_MESH = dict(core_axis_name="c", subcore_axis_name="s")


def _sc_spmm_full():
    d = 128
    cps = EPW // K
    sck = 8
    nsc = cps // sck
    nbuf = 2

    def body(x_hbm, src_hbm, dst_hbm, zeros_hbm, out_hbm,
             acc, idx_s, idx_d, rows, sems):
        cid = lax.axis_index("c")
        sid = lax.axis_index("s")
        wid = cid * NS + sid
        pltpu.sync_copy(zeros_hbm, acc.at[pl.ds(sid * RPS, RPS)])
        pltpu.sync_copy(src_hbm.at[wid], idx_s)
        pltpu.sync_copy(dst_hbm.at[wid], idx_d)
        plsc.subcore_barrier()

        def gather(c, b):
            pltpu.async_copy(x_hbm.at[idx_s.at[c]], rows[b], sems[b])

        def drain_scatter(c, b):
            pltpu.make_async_copy(
                x_hbm.at[idx_s.at[c]], rows[b], sems[b]).wait()
            pltpu.sync_copy(rows[b], acc.at[idx_d.at[c]], add=True)

        for j in range(nbuf):
            gather(j, j % nbuf)

        def step(s, carry):
            c0 = s * sck
            for j in range(sck):
                drain_scatter(c0 + j, j % nbuf)
                gather(c0 + j + nbuf, j % nbuf)
            return carry

        lax.fori_loop(0, nsc - 1, step, 0)
        c0 = (nsc - 1) * sck
        for j in range(sck):
            drain_scatter(c0 + j, j % nbuf)
            if c0 + j + nbuf < cps:
                gather(c0 + j + nbuf, j % nbuf)
        plsc.subcore_barrier()
        pltpu.sync_copy(acc.at[pl.ds(sid * RPS, RPS)],
                        out_hbm.at[cid, pl.ds(sid * RPS, RPS)])

    return pl.kernel(
        body,
        out_type=jax.ShapeDtypeStruct((NC, NPAD, d), jnp.float32),
        mesh=plsc.VectorSubcoreMesh(**_SC_MESH),
        compiler_params=pltpu.CompilerParams(use_tc_tiling_on_sc=False),
        scratch_types=[
            pltpu.VMEM_SHARED((NPAD, d), jnp.float32),
            pltpu.VMEM((cps, K), jnp.int32),
            pltpu.VMEM((cps, K), jnp.int32),
            [pltpu.VMEM((K, d), jnp.float32) for _ in range(nbuf)],
            [pltpu.SemaphoreType.DMA for _ in range(nbuf)],
        ],
    )


def _sc_spmm_cols(dh):
    cps = (EPAD // NS) // K
    sck = 8
    nsc = cps // sck
    nbuf = 4

    def body(x_hbm, src_hbm, dst_hbm, zeros_hbm, out_hbm,
             acc, idx_s, idx_d, rows, sems):
        cid = lax.axis_index("c")
        sid = lax.axis_index("s")
        pltpu.sync_copy(zeros_hbm, acc.at[pl.ds(sid * RPS, RPS)])
        pltpu.sync_copy(src_hbm.at[sid], idx_s)
        pltpu.sync_copy(dst_hbm.at[sid], idx_d)
        plsc.subcore_barrier()

        def gather(c, b):
            pltpu.async_copy(x_hbm.at[cid].at[idx_s.at[c]], rows[b], sems[b])

        def drain_scatter(c, b):
            pltpu.make_async_copy(
                x_hbm.at[cid].at[idx_s.at[c]], rows[b], sems[b]).wait()
            pltpu.sync_copy(rows[b], acc.at[idx_d.at[c]], add=True)

        for j in range(nbuf):
            gather(j, j % nbuf)

        def step(s, carry):
            c0 = s * sck
            for j in range(sck):
                drain_scatter(c0 + j, j % nbuf)
                gather(c0 + j + nbuf, j % nbuf)
            return carry

        lax.fori_loop(0, nsc - 1, step, 0)
        c0 = (nsc - 1) * sck
        for j in range(sck):
            drain_scatter(c0 + j, j % nbuf)
            if c0 + j + nbuf < cps:
                gather(c0 + j + nbuf, j % nbuf)
        plsc.subcore_barrier()
        pltpu.sync_copy(acc.at[pl.ds(sid * RPS, RPS)],
                        out_hbm.at[cid, pl.ds(sid * RPS, RPS)])

    return pl.kernel(
        body,
        out_type=jax.ShapeDtypeStruct((NC, NPAD, dh), jnp.float32),
        mesh=plsc.VectorSubcoreMesh(**_SC_MESH),
        compiler_params=pltpu.CompilerParams(use_tc_tiling_on_sc=False),
        scratch_types=[
            pltpu.VMEM_SHARED((NPAD, dh), jnp.float32),
            pltpu.VMEM((cps, K), jnp.int32),
            pltpu.VMEM((cps, K), jnp.int32),
            [pltpu.VMEM((K, dh), jnp.float32) for _ in range(nbuf)],
            [pltpu.SemaphoreType.DMA for _ in range(nbuf)],
        ],
    )


def _sc_degrees():
    cps = EPW // K
    sck = 8
    nsc = cps // sck

    def body(src_hbm, dst_hbm, ones_hbm, zeros_hbm, out_hbm,
             acc, idx_s, idx_d, ones_s, ones_d, sem):
        cid = lax.axis_index("c")
        sid = lax.axis_index("s")
        wid = cid * NS + sid
        pltpu.sync_copy(ones_hbm.at[0], ones_s)
        pltpu.sync_copy(ones_hbm.at[1], ones_d)
        pltpu.sync_copy(zeros_hbm, acc.at[pl.ds(sid * RPS, RPS)])
        pltpu.sync_copy(src_hbm.at[wid], idx_s)
        pltpu.sync_copy(dst_hbm.at[wid], idx_d)
        plsc.subcore_barrier()

        def issue(c):
            pltpu.async_copy(ones_s, acc.at[idx_s.at[c]], sem, add=True)
            pltpu.async_copy(ones_d, acc.at[idx_d.at[c]], sem, add=True)

        def drain(c):
            pltpu.make_async_copy(ones_s, acc.at[idx_s.at[c]], sem).wait()
            pltpu.make_async_copy(ones_d, acc.at[idx_d.at[c]], sem).wait()

        for j in range(sck):
            issue(j)

        def step(s, carry):
            c0 = s * sck
            for j in range(sck):
                drain(c0 + j)
                issue(c0 + j + sck)
            return carry

        lax.fori_loop(0, nsc - 1, step, 0)
        c0 = (nsc - 1) * sck
        for j in range(sck):
            drain(c0 + j)
        plsc.subcore_barrier()
        pltpu.sync_copy(acc.at[pl.ds(sid * RPS, RPS)],
                        out_hbm.at[cid, pl.ds(sid * RPS, RPS)])

    return pl.kernel(
        body,
        out_type=jax.ShapeDtypeStruct((NC, NPAD, 8), jnp.float32),
        mesh=plsc.VectorSubcoreMesh(**_SC_MESH),
        compiler_params=pltpu.CompilerParams(use_tc_tiling_on_sc=False),
        scratch_types=[
            pltpu.VMEM_SHARED((NPAD, 8), jnp.float32),
            pltpu.VMEM((cps, K), jnp.int32),
            pltpu.VMEM((cps, K), jnp.int32),
            pltpu.VMEM((K, 8), jnp.float32),
            pltpu.VMEM((K, 8), jnp.float32),
            pltpu.SemaphoreType.DMA,
        ],
    )


def _norms(degp_ref):
    deg = degp_ref[0] + degp_ref[1]
    out_deg = deg[:N, 0:1]
    in_deg = deg[:N, 1:2]
    n_src = lax.rsqrt(jnp.where(out_deg > 0, out_deg, 1.0))
    n_dst = lax.rsqrt(jnp.where(in_deg > 0, in_deg, 1.0))
    return n_src, n_dst


def _tc0(degp_ref, feat_ref, x0_ref):
    n_src, _ = _norms(degp_ref)
    x0_ref[:N] = feat_ref[...] * n_src
    x0_ref[N:] = jnp.zeros((NPAD - N, x0_ref.shape[1]), jnp.float32)


def _tc_mid(degp_ref, p_ref, w_ref, b_ref, g_ref, be_ref, out_ref):
    n_src, n_dst = _norms(degp_ref)
    t = (p_ref[0, :N, :] + p_ref[1, :N, :]) * n_dst
    u = jnp.dot(t, w_ref[...], preferred_element_type=jnp.float32) + b_ref[...]
    m = jnp.mean(u, axis=0, keepdims=True)
    c = u - m
    var = jnp.mean(c * c, axis=0, keepdims=True)
    v = c * lax.rsqrt(var + 1e-5) * g_ref[...] + be_ref[...]
    out_ref[:N] = jnp.maximum(v, 0.0) * n_src
    out_ref[N:] = jnp.zeros((NPAD - N, out_ref.shape[1]), jnp.float32)


def _tc_y2(x2_ref, w2p_ref, out_ref):
    y = jnp.dot(x2_ref[:N], w2p_ref[...], preferred_element_type=jnp.float32)
    dh = out_ref.shape[2]
    for c in range(NC):
        out_ref[c, :N] = y[:, c * dh:(c + 1) * dh]
        out_ref[c, N:] = jnp.zeros((NPAD - N, dh), jnp.float32)


def _tc3(degp_ref, q_ref, b2_ref, out_ref):
    _, n_dst = _norms(degp_ref)
    agg = jnp.concatenate([q_ref[0, :N, :], q_ref[1, :N, :]], axis=1)
    out_ref[...] = agg[:, :out_ref.shape[1]] * n_dst + b2_ref[...]


def kernel(feat, edge_index, W0, b0, g0, be0, W1, b1, g1, be1, W2, b2):
    src = edge_index[0]
    dst = edge_index[1]
    d_hid = W0.shape[1]
    n_cls = W2.shape[1]
    dh_c = (-(-n_cls // (8 * NC))) * 8

    zeros_f = jnp.zeros((RPS, d_hid), jnp.float32)
    zeros_c = jnp.zeros((RPS, dh_c), jnp.float32)
    zeros_8 = jnp.zeros((RPS, 8), jnp.float32)
    ones_8 = (jnp.zeros((2, K, 8), jnp.float32)
              .at[0, :, 0].set(1.0).at[1, :, 1].set(1.0))
    w2p = jnp.pad(W2, ((0, 0), (0, NC * dh_c - n_cls)))

    pad_e = jnp.full((EPAD - E,), N, jnp.int32)
    srcp = jnp.concatenate([src, pad_e])
    dstp = jnp.concatenate([dst, pad_e])
    src_w = srcp.reshape(NW, -1, K)
    dst_w = dstp.reshape(NW, -1, K)
    src_s = srcp.reshape(NS, -1, K)
    dst_s = dstp.reshape(NS, -1, K)

    degp = _sc_degrees()(src_w, dst_w, ones_8, zeros_8)

    x0 = pl.pallas_call(
        _tc0, out_shape=jax.ShapeDtypeStruct((NPAD, d_hid), jnp.float32),
    )(degp, feat)

    spmm = _sc_spmm_full()
    Ws = jnp.stack([W0, W1])
    bs = jnp.stack([b0, b1])
    gs = jnp.stack([g0, g1])
    bes = jnp.stack([be0, be1])

    def layer(i, x):
        p = spmm(x, src_w, dst_w, zeros_f)
        return pl.pallas_call(
            _tc_mid, out_shape=jax.ShapeDtypeStruct((NPAD, d_hid), jnp.float32),
        )(degp, p, Ws[i], bs[i], gs[i], bes[i])

    x2 = lax.fori_loop(0, 2, layer, x0)

    y2 = pl.pallas_call(
        _tc_y2, out_shape=jax.ShapeDtypeStruct((NC, NPAD, dh_c), jnp.float32),
    )(x2, w2p)

    q = _sc_spmm_cols(dh_c)(y2, src_s, dst_s, zeros_c)
    out = pl.pallas_call(
        _tc3, out_shape=jax.ShapeDtypeStruct((N, n_cls), jnp.float32),
    )(degp, q, b2)
    return out


# speedup vs baseline: 2.9102x; 2.9102x over previous
import jax
import jax.numpy as jnp
from jax import lax
from jax.experimental import pallas as pl
from jax.experimental.pallas import tpu as pltpu
from jax.experimental.pallas import tpu_sc as plsc

N = 10000
E = 320000
NC = 2
NS = 16
NW = NC * NS
K = 80
NPAD = 10112
RPS = NPAD // NS
EPW = 10240
EPAD = NW * EPW


_SC_MESH = dict(core_axis_name="c", subcore_axis_name="s")


def _sc_spmm_full():
    d = 128
    cps = EPW // K
    sck = 8
    nsc = cps // sck
    nbuf = 2

    def body(x_hbm, src_hbm, dst_hbm, zeros_hbm, out_hbm,
             acc, idx_s, idx_d, rows, sems):
        cid = lax.axis_index("c")
        sid = lax.axis_index("s")
        wid = cid * NS + sid
        pltpu.sync_copy(zeros_hbm, acc.at[pl.ds(sid * RPS, RPS)])
        pltpu.sync_copy(src_hbm.at[wid], idx_s)
        pltpu.sync_copy(dst_hbm.at[wid], idx_d)
        plsc.subcore_barrier()

        def gather(c, b):
            pltpu.async_copy(x_hbm.at[idx_s.at[c]], rows[b], sems[b])

        def drain_scatter(c, b):
            pltpu.make_async_copy(
                x_hbm.at[idx_s.at[c]], rows[b], sems[b]).wait()
            pltpu.sync_copy(rows[b], acc.at[idx_d.at[c]], add=True)

        for j in range(nbuf):
            gather(j, j % nbuf)

        def step(s, carry):
            c0 = s * sck
            for j in range(sck):
                drain_scatter(c0 + j, j % nbuf)
                gather(c0 + j + nbuf, j % nbuf)
            return carry

        lax.fori_loop(0, nsc - 1, step, 0)
        c0 = (nsc - 1) * sck
        for j in range(sck):
            drain_scatter(c0 + j, j % nbuf)
            if c0 + j + nbuf < cps:
                gather(c0 + j + nbuf, j % nbuf)
        plsc.subcore_barrier()
        pltpu.sync_copy(acc.at[pl.ds(sid * RPS, RPS)],
                        out_hbm.at[cid, pl.ds(sid * RPS, RPS)])

    return pl.kernel(
        body,
        out_type=jax.ShapeDtypeStruct((NC, NPAD, d), jnp.float32),
        mesh=plsc.VectorSubcoreMesh(**_SC_MESH),
        compiler_params=pltpu.CompilerParams(use_tc_tiling_on_sc=False),
        scratch_types=[
            pltpu.VMEM_SHARED((NPAD, d), jnp.float32),
            pltpu.VMEM((cps, K), jnp.int32),
            pltpu.VMEM((cps, K), jnp.int32),
            [pltpu.VMEM((K, d), jnp.float32) for _ in range(nbuf)],
            [pltpu.SemaphoreType.DMA for _ in range(nbuf)],
        ],
    )


def _sc_spmm_cols(dh):
    cps = (EPAD // NS) // K
    sck = 8
    nsc = cps // sck
    nbuf = 4

    def body(x_hbm, src_hbm, dst_hbm, zeros_hbm, out_hbm,
             acc, idx_s, idx_d, rows, sems):
        cid = lax.axis_index("c")
        sid = lax.axis_index("s")
        pltpu.sync_copy(zeros_hbm, acc.at[pl.ds(sid * RPS, RPS)])
        pltpu.sync_copy(src_hbm.at[sid], idx_s)
        pltpu.sync_copy(dst_hbm.at[sid], idx_d)
        plsc.subcore_barrier()

        def gather(c, b):
            pltpu.async_copy(x_hbm.at[cid].at[idx_s.at[c]], rows[b], sems[b])

        def drain_scatter(c, b):
            pltpu.make_async_copy(
                x_hbm.at[cid].at[idx_s.at[c]], rows[b], sems[b]).wait()
            pltpu.sync_copy(rows[b], acc.at[idx_d.at[c]], add=True)

        for j in range(nbuf):
            gather(j, j % nbuf)

        def step(s, carry):
            c0 = s * sck
            for j in range(sck):
                drain_scatter(c0 + j, j % nbuf)
                gather(c0 + j + nbuf, j % nbuf)
            return carry

        lax.fori_loop(0, nsc - 1, step, 0)
        c0 = (nsc - 1) * sck
        for j in range(sck):
            drain_scatter(c0 + j, j % nbuf)
            if c0 + j + nbuf < cps:
                gather(c0 + j + nbuf, j % nbuf)
        plsc.subcore_barrier()
        pltpu.sync_copy(acc.at[pl.ds(sid * RPS, RPS)],
                        out_hbm.at[cid, pl.ds(sid * RPS, RPS)])

    return pl.kernel(
        body,
        out_type=jax.ShapeDtypeStruct((NC, NPAD, dh), jnp.float32),
        mesh=plsc.VectorSubcoreMesh(**_SC_MESH),
        compiler_params=pltpu.CompilerParams(use_tc_tiling_on_sc=False),
        scratch_types=[
            pltpu.VMEM_SHARED((NPAD, dh), jnp.float32),
            pltpu.VMEM((cps, K), jnp.int32),
            pltpu.VMEM((cps, K), jnp.int32),
            [pltpu.VMEM((K, dh), jnp.float32) for _ in range(nbuf)],
            [pltpu.SemaphoreType.DMA for _ in range(nbuf)],
        ],
    )


def _sc_degrees():
    cps = EPW // K
    sck = 8
    nsc = cps // sck

    def body(src_hbm, dst_hbm, ones_hbm, zeros_hbm, out_hbm,
             acc, idx_s, idx_d, ones_s, ones_d, sem):
        cid = lax.axis_index("c")
        sid = lax.axis_index("s")
        wid = cid * NS + sid
        pltpu.sync_copy(ones_hbm.at[0], ones_s)
        pltpu.sync_copy(ones_hbm.at[1], ones_d)
        pltpu.sync_copy(zeros_hbm, acc.at[pl.ds(sid * RPS, RPS)])
        pltpu.sync_copy(src_hbm.at[wid], idx_s)
        pltpu.sync_copy(dst_hbm.at[wid], idx_d)
        plsc.subcore_barrier()

        def issue(c):
            pltpu.async_copy(ones_s, acc.at[idx_s.at[c]], sem, add=True)
            pltpu.async_copy(ones_d, acc.at[idx_d.at[c]], sem, add=True)

        def drain(c):
            pltpu.make_async_copy(ones_s, acc.at[idx_s.at[c]], sem).wait()
            pltpu.make_async_copy(ones_d, acc.at[idx_d.at[c]], sem).wait()

        for j in range(sck):
            issue(j)

        def step(s, carry):
            c0 = s * sck
            for j in range(sck):
                drain(c0 + j)
                issue(c0 + j + sck)
            return carry

        lax.fori_loop(0, nsc - 1, step, 0)
        c0 = (nsc - 1) * sck
        for j in range(sck):
            drain(c0 + j)
        plsc.subcore_barrier()
        pltpu.sync_copy(acc.at[pl.ds(sid * RPS, RPS)],
                        out_hbm.at[cid, pl.ds(sid * RPS, RPS)])

    return pl.kernel(
        body,
        out_type=jax.ShapeDtypeStruct((NC, NPAD, 8), jnp.float32),
        mesh=plsc.VectorSubcoreMesh(**_SC_MESH),
        compiler_params=pltpu.CompilerParams(use_tc_tiling_on_sc=False),
        scratch_types=[
            pltpu.VMEM_SHARED((NPAD, 8), jnp.float32),
            pltpu.VMEM((cps, K), jnp.int32),
            pltpu.VMEM((cps, K), jnp.int32),
            pltpu.VMEM((K, 8), jnp.float32),
            pltpu.VMEM((K, 8), jnp.float32),
            pltpu.SemaphoreType.DMA,
        ],
    )


def _norms(degp_ref):
    deg = degp_ref[0] + degp_ref[1]
    out_deg = deg[:N, 0:1]
    in_deg = deg[:N, 1:2]
    n_src = lax.rsqrt(jnp.where(out_deg > 0, out_deg, 1.0))
    n_dst = lax.rsqrt(jnp.where(in_deg > 0, in_deg, 1.0))
    return n_src, n_dst


def _tc0(degp_ref, feat_ref, x0_ref):
    n_src, _ = _norms(degp_ref)
    x0_ref[:N] = feat_ref[...] * n_src
    x0_ref[N:] = jnp.zeros((NPAD - N, x0_ref.shape[1]), jnp.float32)


def _tc_mid(degp_ref, p_ref, w_ref, b_ref, g_ref, be_ref, out_ref):
    n_src, n_dst = _norms(degp_ref)
    t = (p_ref[0, :N, :] + p_ref[1, :N, :]) * n_dst
    u = jnp.dot(t, w_ref[...], preferred_element_type=jnp.float32) + b_ref[...]
    m = jnp.mean(u, axis=0, keepdims=True)
    c = u - m
    var = jnp.mean(c * c, axis=0, keepdims=True)
    v = c * lax.rsqrt(var + 1e-5) * g_ref[...] + be_ref[...]
    out_ref[:N] = jnp.maximum(v, 0.0) * n_src
    out_ref[N:] = jnp.zeros((NPAD - N, out_ref.shape[1]), jnp.float32)


def _tc_y2(x2_ref, w2p_ref, out_ref):
    y = jnp.dot(x2_ref[:N], w2p_ref[...], preferred_element_type=jnp.float32)
    dh = out_ref.shape[2]
    for c in range(NC):
        out_ref[c, :N] = y[:, c * dh:(c + 1) * dh]
        out_ref[c, N:] = jnp.zeros((NPAD - N, dh), jnp.float32)


def _tc3(degp_ref, q_ref, b2_ref, out_ref):
    _, n_dst = _norms(degp_ref)
    agg = jnp.concatenate([q_ref[0, :N, :], q_ref[1, :N, :]], axis=1)
    out_ref[...] = agg[:, :out_ref.shape[1]] * n_dst + b2_ref[...]


def kernel(feat, edge_index, W0, b0, g0, be0, W1, b1, g1, be1, W2, b2):
    src = edge_index[0]
    dst = edge_index[1]
    d_hid = W0.shape[1]
    n_cls = W2.shape[1]
    dh_c = (-(-n_cls // (8 * NC))) * 8

    zeros_f = jnp.zeros((RPS, d_hid), jnp.float32)
    zeros_c = jnp.zeros((RPS, dh_c), jnp.float32)
    zeros_8 = jnp.zeros((RPS, 8), jnp.float32)
    ones_8 = (jnp.zeros((2, K, 8), jnp.float32)
              .at[0, :, 0].set(1.0).at[1, :, 1].set(1.0))
    w2p = jnp.pad(W2, ((0, 0), (0, NC * dh_c - n_cls)))

    pad_e = N + (jnp.arange(EPAD - E, dtype=jnp.int32) % (NPAD - N))
    srcp = jnp.concatenate([src, pad_e])
    dstp = jnp.concatenate([dst, pad_e])
    src_w = srcp.reshape(NW, -1, K)
    dst_w = dstp.reshape(NW, -1, K)
    src_s = srcp.reshape(NS, -1, K)
    dst_s = dstp.reshape(NS, -1, K)

    degp = _sc_degrees()(src_w, dst_w, ones_8, zeros_8)

    x0 = pl.pallas_call(
        _tc0, out_shape=jax.ShapeDtypeStruct((NPAD, d_hid), jnp.float32),
    )(degp, feat)

    spmm = _sc_spmm_full()
    Ws = jnp.stack([W0, W1])
    bs = jnp.stack([b0, b1])
    gs = jnp.stack([g0, g1])
    bes = jnp.stack([be0, be1])

    def layer(i, x):
        p = spmm(x, src_w, dst_w, zeros_f)
        return pl.pallas_call(
            _tc_mid, out_shape=jax.ShapeDtypeStruct((NPAD, d_hid), jnp.float32),
        )(degp, p, Ws[i], bs[i], gs[i], bes[i])

    x2 = lax.fori_loop(0, 2, layer, x0)

    y2 = pl.pallas_call(
        _tc_y2, out_shape=jax.ShapeDtypeStruct((NC, NPAD, dh_c), jnp.float32),
    )(x2, w2p)

    q = _sc_spmm_cols(dh_c)(y2, src_s, dst_s, zeros_c)
    out = pl.pallas_call(
        _tc3, out_shape=jax.ShapeDtypeStruct((N, n_cls), jnp.float32),
    )(degp, q, b2)
    return out


# speedup vs baseline: 3.4909x; 1.1996x over previous
import jax
import jax.numpy as jnp
from jax import lax
from jax.experimental import pallas as pl
from jax.experimental.pallas import tpu as pltpu
from jax.experimental.pallas import tpu_sc as plsc

N = 10000
E = 320000
NC = 2
NS = 16
NW = NC * NS
K = 80
NPAD = 10112
RPS = NPAD // NS
EPW = 10240
EPAD = NW * EPW


_SC_MESH = dict(core_axis_name="c", subcore_axis_name="s")


def _sc_spmm_full():
    d = 128
    cps = EPW // K
    sck = 8
    nsc = cps // sck
    nbuf = 4

    def body(x_hbm, src_hbm, dst_hbm, zeros_hbm, out_hbm,
             acc, idx_s, idx_d, rows, sems, isem):
        cid = lax.axis_index("c")
        sid = lax.axis_index("s")
        wid = cid * NS + sid
        pltpu.sync_copy(zeros_hbm, acc.at[pl.ds(sid * RPS, RPS)])

        def load_idx(t, half):
            sl = pl.ds(t * sck, sck)
            hl = pl.ds(half * sck, sck)
            pltpu.async_copy(src_hbm.at[wid, sl], idx_s.at[hl], isem)
            pltpu.async_copy(dst_hbm.at[wid, sl], idx_d.at[hl], isem)

        def wait_idx(t, half):
            sl = pl.ds(t * sck, sck)
            hl = pl.ds(half * sck, sck)
            pltpu.make_async_copy(src_hbm.at[wid, sl], idx_s.at[hl], isem).wait()
            pltpu.make_async_copy(dst_hbm.at[wid, sl], idx_d.at[hl], isem).wait()

        def gather(r, b):
            pltpu.async_copy(x_hbm.at[idx_s.at[r]], rows[b], sems[b])

        def drain_scatter(r, b):
            pltpu.make_async_copy(
                x_hbm.at[idx_s.at[r]], rows[b], sems[b]).wait()
            pltpu.sync_copy(rows[b], acc.at[idx_d.at[r]], add=True)

        load_idx(0, 0)
        wait_idx(0, 0)
        load_idx(1, 1)
        plsc.subcore_barrier()
        for j in range(nbuf):
            gather(j, j)

        def step_body(s, issue_next):
            cur = (s % 2) * sck
            nxt = ((s + 1) % 2) * sck
            for j in range(nbuf):
                drain_scatter(cur + j, j)
                gather(cur + j + nbuf, j)
            wait_idx(s + 1, (s + 1) % 2)
            for j in range(nbuf, sck):
                drain_scatter(cur + j, j - nbuf)
                gather(nxt + j - nbuf, j - nbuf)
            if issue_next:
                load_idx(s + 2, s % 2)

        lax.fori_loop(0, nsc - 2,
                      lambda s, c: (step_body(s, True), c)[1], 0)
        step_body(nsc - 2, False)
        cur = ((nsc - 1) % 2) * sck
        for j in range(sck):
            drain_scatter(cur + j, j % nbuf)
            if j + nbuf < sck:
                gather(cur + j + nbuf, j % nbuf)
        plsc.subcore_barrier()
        pltpu.sync_copy(acc.at[pl.ds(sid * RPS, RPS)],
                        out_hbm.at[cid, pl.ds(sid * RPS, RPS)])

    return pl.kernel(
        body,
        out_type=jax.ShapeDtypeStruct((NC, NPAD, d), jnp.float32),
        mesh=plsc.VectorSubcoreMesh(**_SC_MESH),
        compiler_params=pltpu.CompilerParams(use_tc_tiling_on_sc=False),
        scratch_types=[
            pltpu.VMEM_SHARED((NPAD, d), jnp.float32),
            pltpu.VMEM((2 * sck, K), jnp.int32),
            pltpu.VMEM((2 * sck, K), jnp.int32),
            [pltpu.VMEM((K, d), jnp.float32) for _ in range(nbuf)],
            [pltpu.SemaphoreType.DMA for _ in range(nbuf)],
            pltpu.SemaphoreType.DMA,
        ],
    )


def _sc_spmm_cols(dh):
    cps = (EPAD // NS) // K
    sck = 8
    nsc = cps // sck
    nbuf = 8

    def body(x_hbm, src_hbm, dst_hbm, zeros_hbm, out_hbm,
             acc, idx_s, idx_d, rows, sems):
        cid = lax.axis_index("c")
        sid = lax.axis_index("s")
        pltpu.sync_copy(zeros_hbm, acc.at[pl.ds(sid * RPS, RPS)])
        pltpu.sync_copy(src_hbm.at[sid], idx_s)
        pltpu.sync_copy(dst_hbm.at[sid], idx_d)
        plsc.subcore_barrier()

        def gather(c, b):
            pltpu.async_copy(x_hbm.at[cid].at[idx_s.at[c]], rows[b], sems[b])

        def drain_scatter(c, b):
            pltpu.make_async_copy(
                x_hbm.at[cid].at[idx_s.at[c]], rows[b], sems[b]).wait()
            pltpu.sync_copy(rows[b], acc.at[idx_d.at[c]], add=True)

        for j in range(nbuf):
            gather(j, j % nbuf)

        def step(s, carry):
            c0 = s * sck
            for j in range(sck):
                drain_scatter(c0 + j, j % nbuf)
                gather(c0 + j + nbuf, j % nbuf)
            return carry

        lax.fori_loop(0, nsc - 1, step, 0)
        c0 = (nsc - 1) * sck
        for j in range(sck):
            drain_scatter(c0 + j, j % nbuf)
            if c0 + j + nbuf < cps:
                gather(c0 + j + nbuf, j % nbuf)
        plsc.subcore_barrier()
        pltpu.sync_copy(acc.at[pl.ds(sid * RPS, RPS)],
                        out_hbm.at[cid, pl.ds(sid * RPS, RPS)])

    return pl.kernel(
        body,
        out_type=jax.ShapeDtypeStruct((NC, NPAD, dh), jnp.float32),
        mesh=plsc.VectorSubcoreMesh(**_SC_MESH),
        compiler_params=pltpu.CompilerParams(use_tc_tiling_on_sc=False),
        scratch_types=[
            pltpu.VMEM_SHARED((NPAD, dh), jnp.float32),
            pltpu.VMEM((cps, K), jnp.int32),
            pltpu.VMEM((cps, K), jnp.int32),
            [pltpu.VMEM((K, dh), jnp.float32) for _ in range(nbuf)],
            [pltpu.SemaphoreType.DMA for _ in range(nbuf)],
        ],
    )


def _sc_degrees():
    cps = EPW // K
    sck = 8
    nsc = cps // sck

    def body(src_hbm, dst_hbm, ones_hbm, zeros_hbm, out_hbm,
             acc, idx_s, idx_d, ones_s, ones_d, sem):
        cid = lax.axis_index("c")
        sid = lax.axis_index("s")
        wid = cid * NS + sid
        pltpu.sync_copy(ones_hbm.at[0], ones_s)
        pltpu.sync_copy(ones_hbm.at[1], ones_d)
        pltpu.sync_copy(zeros_hbm, acc.at[pl.ds(sid * RPS, RPS)])
        pltpu.sync_copy(src_hbm.at[wid], idx_s)
        pltpu.sync_copy(dst_hbm.at[wid], idx_d)
        plsc.subcore_barrier()

        def issue(c):
            pltpu.async_copy(ones_s, acc.at[idx_s.at[c]], sem, add=True)
            pltpu.async_copy(ones_d, acc.at[idx_d.at[c]], sem, add=True)

        def drain(c):
            pltpu.make_async_copy(ones_s, acc.at[idx_s.at[c]], sem).wait()
            pltpu.make_async_copy(ones_d, acc.at[idx_d.at[c]], sem).wait()

        for j in range(sck):
            issue(j)

        def step(s, carry):
            c0 = s * sck
            for j in range(sck):
                drain(c0 + j)
                issue(c0 + j + sck)
            return carry

        lax.fori_loop(0, nsc - 1, step, 0)
        c0 = (nsc - 1) * sck
        for j in range(sck):
            drain(c0 + j)
        plsc.subcore_barrier()
        pltpu.sync_copy(acc.at[pl.ds(sid * RPS, RPS)],
                        out_hbm.at[cid, pl.ds(sid * RPS, RPS)])

    return pl.kernel(
        body,
        out_type=jax.ShapeDtypeStruct((NC, NPAD, 8), jnp.float32),
        mesh=plsc.VectorSubcoreMesh(**_SC_MESH),
        compiler_params=pltpu.CompilerParams(use_tc_tiling_on_sc=False),
        scratch_types=[
            pltpu.VMEM_SHARED((NPAD, 8), jnp.float32),
            pltpu.VMEM((cps, K), jnp.int32),
            pltpu.VMEM((cps, K), jnp.int32),
            pltpu.VMEM((K, 8), jnp.float32),
            pltpu.VMEM((K, 8), jnp.float32),
            pltpu.SemaphoreType.DMA,
        ],
    )


def _norms(degp_ref):
    deg = degp_ref[0] + degp_ref[1]
    out_deg = deg[:N, 0:1]
    in_deg = deg[:N, 1:2]
    n_src = lax.rsqrt(jnp.where(out_deg > 0, out_deg, 1.0))
    n_dst = lax.rsqrt(jnp.where(in_deg > 0, in_deg, 1.0))
    return n_src, n_dst


def _tc0(degp_ref, feat_ref, x0_ref):
    n_src, _ = _norms(degp_ref)
    x0_ref[:N] = feat_ref[...] * n_src
    x0_ref[N:] = jnp.zeros((NPAD - N, x0_ref.shape[1]), jnp.float32)


def _tc_mid(degp_ref, p_ref, w_ref, b_ref, g_ref, be_ref, out_ref):
    n_src, n_dst = _norms(degp_ref)
    t = (p_ref[0, :N, :] + p_ref[1, :N, :]) * n_dst
    u = jnp.dot(t, w_ref[...], preferred_element_type=jnp.float32) + b_ref[...]
    m = jnp.mean(u, axis=0, keepdims=True)
    c = u - m
    var = jnp.mean(c * c, axis=0, keepdims=True)
    v = c * lax.rsqrt(var + 1e-5) * g_ref[...] + be_ref[...]
    out_ref[:N] = jnp.maximum(v, 0.0) * n_src
    out_ref[N:] = jnp.zeros((NPAD - N, out_ref.shape[1]), jnp.float32)


def _tc_y2(x2_ref, w2p_ref, out_ref):
    y = jnp.dot(x2_ref[:N], w2p_ref[...], preferred_element_type=jnp.float32)
    dh = out_ref.shape[2]
    for c in range(NC):
        out_ref[c, :N] = y[:, c * dh:(c + 1) * dh]
        out_ref[c, N:] = jnp.zeros((NPAD - N, dh), jnp.float32)


def _tc3(degp_ref, q_ref, b2_ref, out_ref):
    _, n_dst = _norms(degp_ref)
    agg = jnp.concatenate([q_ref[0, :N, :], q_ref[1, :N, :]], axis=1)
    out_ref[...] = agg[:, :out_ref.shape[1]] * n_dst + b2_ref[...]


def kernel(feat, edge_index, W0, b0, g0, be0, W1, b1, g1, be1, W2, b2):
    src = edge_index[0]
    dst = edge_index[1]
    d_hid = W0.shape[1]
    n_cls = W2.shape[1]
    dh_c = (-(-n_cls // (8 * NC))) * 8

    zeros_f = jnp.zeros((RPS, d_hid), jnp.float32)
    zeros_c = jnp.zeros((RPS, dh_c), jnp.float32)
    zeros_8 = jnp.zeros((RPS, 8), jnp.float32)
    ones_8 = (jnp.zeros((2, K, 8), jnp.float32)
              .at[0, :, 0].set(1.0).at[1, :, 1].set(1.0))
    w2p = jnp.pad(W2, ((0, 0), (0, NC * dh_c - n_cls)))

    pad_e = N + (jnp.arange(EPAD - E, dtype=jnp.int32) % (NPAD - N))
    srcp = jnp.concatenate([src, pad_e])
    dstp = jnp.concatenate([dst, pad_e])
    src_w = srcp.reshape(NW, -1, K)
    dst_w = dstp.reshape(NW, -1, K)
    src_s = srcp.reshape(NS, -1, K)
    dst_s = dstp.reshape(NS, -1, K)

    degp = _sc_degrees()(src_w, dst_w, ones_8, zeros_8)

    x0 = pl.pallas_call(
        _tc0, out_shape=jax.ShapeDtypeStruct((NPAD, d_hid), jnp.float32),
    )(degp, feat)

    spmm = _sc_spmm_full()
    Ws = jnp.stack([W0, W1])
    bs = jnp.stack([b0, b1])
    gs = jnp.stack([g0, g1])
    bes = jnp.stack([be0, be1])

    def layer(i, x):
        p = spmm(x, src_w, dst_w, zeros_f)
        return pl.pallas_call(
            _tc_mid, out_shape=jax.ShapeDtypeStruct((NPAD, d_hid), jnp.float32),
        )(degp, p, Ws[i], bs[i], gs[i], bes[i])

    x2 = lax.fori_loop(0, 2, layer, x0)

    y2 = pl.pallas_call(
        _tc_y2, out_shape=jax.ShapeDtypeStruct((NC, NPAD, dh_c), jnp.float32),
    )(x2, w2p)

    q = _sc_spmm_cols(dh_c)(y2, src_s, dst_s, zeros_c)
    out = pl.pallas_call(
        _tc3, out_shape=jax.ShapeDtypeStruct((N, n_cls), jnp.float32),
    )(degp, q, b2)
    return out


# speedup vs baseline: 3.5030x; 1.0034x over previous
import jax
import jax.numpy as jnp
from jax import lax
from jax.experimental import pallas as pl
from jax.experimental.pallas import tpu as pltpu
from jax.experimental.pallas import tpu_sc as plsc

N = 10000
E = 320000
NC = 2
NS = 16
NW = NC * NS
K = 80
NPAD = 10112
RPS = NPAD // NS
EPW = 10240
EPAD = NW * EPW


_SC_MESH = dict(core_axis_name="c", subcore_axis_name="s")


def _sc_spmm_full():
    d = 128
    cps = EPW // K
    sck = 8
    nsc = cps // sck
    nbuf = 4

    def body(x_hbm, src_hbm, dst_hbm, zeros_hbm, out_hbm,
             acc, idx_s, idx_d, rows, sems, isem):
        cid = lax.axis_index("c")
        sid = lax.axis_index("s")
        wid = cid * NS + sid
        pltpu.sync_copy(zeros_hbm, acc.at[pl.ds(sid * RPS, RPS)])

        def load_idx(t, half):
            sl = pl.ds(t * sck, sck)
            hl = pl.ds(half * sck, sck)
            pltpu.async_copy(src_hbm.at[wid, sl], idx_s.at[hl], isem)
            pltpu.async_copy(dst_hbm.at[wid, sl], idx_d.at[hl], isem)

        def wait_idx(t, half):
            sl = pl.ds(t * sck, sck)
            hl = pl.ds(half * sck, sck)
            pltpu.make_async_copy(src_hbm.at[wid, sl], idx_s.at[hl], isem).wait()
            pltpu.make_async_copy(dst_hbm.at[wid, sl], idx_d.at[hl], isem).wait()

        def gather(r, b):
            pltpu.async_copy(x_hbm.at[idx_s.at[r]], rows[b], sems[b])

        def drain_scatter(r, b):
            pltpu.make_async_copy(
                x_hbm.at[idx_s.at[r]], rows[b], sems[b]).wait()
            pltpu.sync_copy(rows[b], acc.at[idx_d.at[r]], add=True)

        load_idx(0, 0)
        wait_idx(0, 0)
        load_idx(1, 1)
        plsc.subcore_barrier()
        for j in range(nbuf):
            gather(j, j)

        def step_body(s, issue_next):
            cur = (s % 2) * sck
            nxt = ((s + 1) % 2) * sck
            for j in range(nbuf):
                drain_scatter(cur + j, j)
                gather(cur + j + nbuf, j)
            wait_idx(s + 1, (s + 1) % 2)
            for j in range(nbuf, sck):
                drain_scatter(cur + j, j - nbuf)
                gather(nxt + j - nbuf, j - nbuf)
            if issue_next:
                load_idx(s + 2, s % 2)

        lax.fori_loop(0, nsc - 2,
                      lambda s, c: (step_body(s, True), c)[1], 0)
        step_body(nsc - 2, False)
        cur = ((nsc - 1) % 2) * sck
        for j in range(sck):
            drain_scatter(cur + j, j % nbuf)
            if j + nbuf < sck:
                gather(cur + j + nbuf, j % nbuf)
        plsc.subcore_barrier()
        pltpu.sync_copy(acc.at[pl.ds(sid * RPS, RPS)],
                        out_hbm.at[cid, pl.ds(sid * RPS, RPS)])

    return pl.kernel(
        body,
        out_type=jax.ShapeDtypeStruct((NC, NPAD, d), jnp.float32),
        mesh=plsc.VectorSubcoreMesh(**_SC_MESH),
        scratch_types=[
            pltpu.VMEM_SHARED((NPAD, d), jnp.float32),
            pltpu.VMEM((2 * sck, K), jnp.int32),
            pltpu.VMEM((2 * sck, K), jnp.int32),
            [pltpu.VMEM((K, d), jnp.float32) for _ in range(nbuf)],
            [pltpu.SemaphoreType.DMA for _ in range(nbuf)],
            pltpu.SemaphoreType.DMA,
        ],
    )


def _sc_spmm_cols(dh):
    cps = (EPAD // NS) // K
    sck = 8
    nsc = cps // sck
    nbuf = 8

    def body(x_hbm, src_hbm, dst_hbm, zeros_hbm, out_hbm,
             acc, idx_s, idx_d, rows, sems):
        cid = lax.axis_index("c")
        sid = lax.axis_index("s")
        pltpu.sync_copy(zeros_hbm, acc.at[pl.ds(sid * RPS, RPS)])
        pltpu.sync_copy(src_hbm.at[sid], idx_s)
        pltpu.sync_copy(dst_hbm.at[sid], idx_d)
        plsc.subcore_barrier()

        def gather(c, b):
            pltpu.async_copy(x_hbm.at[cid].at[idx_s.at[c]], rows[b], sems[b])

        def drain_scatter(c, b):
            pltpu.make_async_copy(
                x_hbm.at[cid].at[idx_s.at[c]], rows[b], sems[b]).wait()
            pltpu.sync_copy(rows[b], acc.at[idx_d.at[c]], add=True)

        for j in range(nbuf):
            gather(j, j % nbuf)

        def step(s, carry):
            c0 = s * sck
            for j in range(sck):
                drain_scatter(c0 + j, j % nbuf)
                gather(c0 + j + nbuf, j % nbuf)
            return carry

        lax.fori_loop(0, nsc - 1, step, 0)
        c0 = (nsc - 1) * sck
        for j in range(sck):
            drain_scatter(c0 + j, j % nbuf)
            if c0 + j + nbuf < cps:
                gather(c0 + j + nbuf, j % nbuf)
        plsc.subcore_barrier()
        pltpu.sync_copy(acc.at[pl.ds(sid * RPS, RPS)],
                        out_hbm.at[cid, pl.ds(sid * RPS, RPS)])

    return pl.kernel(
        body,
        out_type=jax.ShapeDtypeStruct((NC, NPAD, dh), jnp.float32),
        mesh=plsc.VectorSubcoreMesh(**_SC_MESH),
        compiler_params=pltpu.CompilerParams(use_tc_tiling_on_sc=False),
        scratch_types=[
            pltpu.VMEM_SHARED((NPAD, dh), jnp.float32),
            pltpu.VMEM((cps, K), jnp.int32),
            pltpu.VMEM((cps, K), jnp.int32),
            [pltpu.VMEM((K, dh), jnp.float32) for _ in range(nbuf)],
            [pltpu.SemaphoreType.DMA for _ in range(nbuf)],
        ],
    )


def _sc_degrees():
    cps = EPW // K
    sck = 8
    nsc = cps // sck

    def body(src_hbm, dst_hbm, ones_hbm, zeros_hbm, out_hbm,
             acc, idx_s, idx_d, ones_s, ones_d, sem):
        cid = lax.axis_index("c")
        sid = lax.axis_index("s")
        wid = cid * NS + sid
        pltpu.sync_copy(ones_hbm.at[0], ones_s)
        pltpu.sync_copy(ones_hbm.at[1], ones_d)
        pltpu.sync_copy(zeros_hbm, acc.at[pl.ds(sid * RPS, RPS)])
        pltpu.sync_copy(src_hbm.at[wid], idx_s)
        pltpu.sync_copy(dst_hbm.at[wid], idx_d)
        plsc.subcore_barrier()

        def issue(c):
            pltpu.async_copy(ones_s, acc.at[idx_s.at[c]], sem, add=True)
            pltpu.async_copy(ones_d, acc.at[idx_d.at[c]], sem, add=True)

        def drain(c):
            pltpu.make_async_copy(ones_s, acc.at[idx_s.at[c]], sem).wait()
            pltpu.make_async_copy(ones_d, acc.at[idx_d.at[c]], sem).wait()

        for j in range(sck):
            issue(j)

        def step(s, carry):
            c0 = s * sck
            for j in range(sck):
                drain(c0 + j)
                issue(c0 + j + sck)
            return carry

        lax.fori_loop(0, nsc - 1, step, 0)
        c0 = (nsc - 1) * sck
        for j in range(sck):
            drain(c0 + j)
        plsc.subcore_barrier()
        pltpu.sync_copy(acc.at[pl.ds(sid * RPS, RPS)],
                        out_hbm.at[cid, pl.ds(sid * RPS, RPS)])

    return pl.kernel(
        body,
        out_type=jax.ShapeDtypeStruct((NC, NPAD, 8), jnp.float32),
        mesh=plsc.VectorSubcoreMesh(**_SC_MESH),
        compiler_params=pltpu.CompilerParams(use_tc_tiling_on_sc=False),
        scratch_types=[
            pltpu.VMEM_SHARED((NPAD, 8), jnp.float32),
            pltpu.VMEM((cps, K), jnp.int32),
            pltpu.VMEM((cps, K), jnp.int32),
            pltpu.VMEM((K, 8), jnp.float32),
            pltpu.VMEM((K, 8), jnp.float32),
            pltpu.SemaphoreType.DMA,
        ],
    )


def _norms(degp_ref):
    deg = degp_ref[0] + degp_ref[1]
    out_deg = deg[:N, 0:1]
    in_deg = deg[:N, 1:2]
    n_src = lax.rsqrt(jnp.where(out_deg > 0, out_deg, 1.0))
    n_dst = lax.rsqrt(jnp.where(in_deg > 0, in_deg, 1.0))
    return n_src, n_dst


def _tc0(degp_ref, feat_ref, x0_ref):
    n_src, _ = _norms(degp_ref)
    x0_ref[:N] = feat_ref[...] * n_src
    x0_ref[N:] = jnp.zeros((NPAD - N, x0_ref.shape[1]), jnp.float32)


def _tc_mid(degp_ref, p_ref, w_ref, b_ref, g_ref, be_ref, out_ref):
    n_src, n_dst = _norms(degp_ref)
    t = (p_ref[0, :N, :] + p_ref[1, :N, :]) * n_dst
    u = jnp.dot(t, w_ref[...], preferred_element_type=jnp.float32) + b_ref[...]
    m = jnp.mean(u, axis=0, keepdims=True)
    c = u - m
    var = jnp.mean(c * c, axis=0, keepdims=True)
    v = c * lax.rsqrt(var + 1e-5) * g_ref[...] + be_ref[...]
    out_ref[:N] = jnp.maximum(v, 0.0) * n_src
    out_ref[N:] = jnp.zeros((NPAD - N, out_ref.shape[1]), jnp.float32)


def _tc_y2(x2_ref, w2p_ref, out_ref):
    y = jnp.dot(x2_ref[:N], w2p_ref[...], preferred_element_type=jnp.float32)
    dh = out_ref.shape[2]
    for c in range(NC):
        out_ref[c, :N] = y[:, c * dh:(c + 1) * dh]
        out_ref[c, N:] = jnp.zeros((NPAD - N, dh), jnp.float32)


def _tc3(degp_ref, q_ref, b2_ref, out_ref):
    _, n_dst = _norms(degp_ref)
    agg = jnp.concatenate([q_ref[0, :N, :], q_ref[1, :N, :]], axis=1)
    out_ref[...] = agg[:, :out_ref.shape[1]] * n_dst + b2_ref[...]


def kernel(feat, edge_index, W0, b0, g0, be0, W1, b1, g1, be1, W2, b2):
    src = edge_index[0]
    dst = edge_index[1]
    d_hid = W0.shape[1]
    n_cls = W2.shape[1]
    dh_c = (-(-n_cls // (8 * NC))) * 8

    zeros_f = jnp.zeros((RPS, d_hid), jnp.float32)
    zeros_c = jnp.zeros((RPS, dh_c), jnp.float32)
    zeros_8 = jnp.zeros((RPS, 8), jnp.float32)
    ones_8 = (jnp.zeros((2, K, 8), jnp.float32)
              .at[0, :, 0].set(1.0).at[1, :, 1].set(1.0))
    w2p = jnp.pad(W2, ((0, 0), (0, NC * dh_c - n_cls)))

    pad_e = N + (jnp.arange(EPAD - E, dtype=jnp.int32) % (NPAD - N))
    srcp = jnp.concatenate([src, pad_e])
    dstp = jnp.concatenate([dst, pad_e])
    src_w = srcp.reshape(NW, -1, K)
    dst_w = dstp.reshape(NW, -1, K)
    src_s = srcp.reshape(NS, -1, K)
    dst_s = dstp.reshape(NS, -1, K)

    degp = _sc_degrees()(src_w, dst_w, ones_8, zeros_8)

    x0 = pl.pallas_call(
        _tc0, out_shape=jax.ShapeDtypeStruct((NPAD, d_hid), jnp.float32),
    )(degp, feat)

    spmm = _sc_spmm_full()
    Ws = jnp.stack([W0, W1])
    bs = jnp.stack([b0, b1])
    gs = jnp.stack([g0, g1])
    bes = jnp.stack([be0, be1])

    def layer(i, x):
        p = spmm(x, src_w, dst_w, zeros_f)
        return pl.pallas_call(
            _tc_mid, out_shape=jax.ShapeDtypeStruct((NPAD, d_hid), jnp.float32),
        )(degp, p, Ws[i], bs[i], gs[i], bes[i])

    x2 = lax.fori_loop(0, 2, layer, x0)

    y2 = pl.pallas_call(
        _tc_y2, out_shape=jax.ShapeDtypeStruct((NC, NPAD, dh_c), jnp.float32),
    )(x2, w2p)

    q = _sc_spmm_cols(dh_c)(y2, src_s, dst_s, zeros_c)
    out = pl.pallas_call(
        _tc3, out_shape=jax.ShapeDtypeStruct((N, n_cls), jnp.float32),
    )(degp, q, b2)
    return out
